# baseline (speedup 1.0000x reference)
import dataclasses
import functools

import jax
import jax.numpy as jnp
from jax import lax
from jax.experimental import pallas as pl
from jax.experimental.pallas import tpu as pltpu
from jax.experimental.pallas import tpu_sc as plsc

N_NODES = 10000
N_PAD = 10240
D = 256
DH = 128
DT = 144
E = 160000
CHUNK = 128
NCH = 84
NSUB = 16
NCORE = 2
EPW = NCH * CHUNK
E_PAD = EPW * NSUB
Z_RATIO = 0.8
GN_EPS = 1e-9

_HIGH = lax.Precision.HIGHEST


def _tableA_body(x_ref, m_ref, wt_ref, bt_ref, out_ref):
    x = x_ref[...]
    h = jnp.dot(x, wt_ref[...], precision=_HIGH) + bt_ref[...]
    h = jnp.maximum(h, 0.0)
    h1 = h[:, :D]
    h0 = h[:, D:]
    m = m_ref[...]
    c1 = (1.0 - Z_RATIO) + (2.0 * Z_RATIO - 1.0) * m
    xm = c1 * h1 + (1.0 - c1) * h0
    B = x.shape[0]
    it = lax.broadcasted_iota(jnp.int32, (B, 16), 1)
    pad = jnp.where(it == 0, 1.0, 0.0).astype(jnp.float32)
    out_ref[0, :, :DH] = xm[:, :DH]
    out_ref[0, :, DH:DT] = pad
    out_ref[1, :, :DH] = xm[:, DH:]
    out_ref[1, :, DH:DT] = pad


def _build_table(x_, maskf, Wt, bt):
    B = 1000
    grid = (N_NODES // B,)
    return pl.pallas_call(
        _tableA_body,
        grid=grid,
        in_specs=[
            pl.BlockSpec((B, D), lambda i: (i, 0)),
            pl.BlockSpec((B, 1), lambda i: (i, 0)),
            pl.BlockSpec((D, 2 * D), lambda i: (0, 0)),
            pl.BlockSpec((1, 2 * D), lambda i: (0, 0)),
        ],
        out_specs=pl.BlockSpec((2, B, DT), lambda i: (0, i, 0)),
        out_shape=jax.ShapeDtypeStruct((2, N_NODES, DT), jnp.float32),
    )(x_, maskf, Wt, bt)


def _sc_spmm(table, colp, rowp, wp):
    mesh = plsc.VectorSubcoreMesh(core_axis_name="c", subcore_axis_name="s")
    cp = pltpu.CompilerParams()
    if "needs_layout_passes" in pltpu.CompilerParams.__dataclass_fields__:
        cp = dataclasses.replace(cp, needs_layout_passes=False)
    if "use_tc_tiling_on_sc" in pltpu.CompilerParams.__dataclass_fields__:
        cp = dataclasses.replace(cp, use_tc_tiling_on_sc=False)

    @functools.partial(
        pl.kernel,
        out_type=jax.ShapeDtypeStruct((NCORE, N_PAD, DT), jnp.float32),
        mesh=mesh,
        compiler_params=cp,
        scratch_types=[
            pltpu.VMEM((3, CHUNK), jnp.int32),
            pltpu.VMEM((3, CHUNK), jnp.int32),
            pltpu.VMEM((3, CHUNK), jnp.float32),
            pltpu.VMEM((CHUNK, DT), jnp.float32),
            pltpu.VMEM((CHUNK, DT), jnp.float32),
            pltpu.VMEM_SHARED((N_PAD, DT), jnp.float32),
            pltpu.SemaphoreType.DMA,
            pltpu.SemaphoreType.DMA,
            pltpu.SemaphoreType.DMA,
            pltpu.SemaphoreType.DMA,
            pltpu.SemaphoreType.DMA,
        ],
    )
    def spmm(table_hbm, col_hbm, row_hbm, w_hbm, out_hbm,
             colb, rowb, wb, r0, r1, acc_sh,
             gs0, gs1, is0, is1, is2):
        cid = lax.axis_index("c")
        sid = lax.axis_index("s")
        rows = [r0, r1]
        gsems = [gs0, gs1]
        isems = [is0, is1, is2]
        zero16 = jnp.zeros((16,), jnp.float32)
        off = cid * N_NODES

        @pl.loop(0, CHUNK)
        def _(r):
            for j in range(DT // 16):
                r0[r, pl.ds(j * 16, 16)] = zero16

        stripe = sid * 640

        @pl.loop(0, 640, step=CHUNK)
        def _(r):
            pltpu.sync_copy(r0, acc_sh.at[pl.ds(stripe + r, CHUNK)])

        plsc.subcore_barrier()

        def idx_copies(g, s):
            return [
                pltpu.make_async_copy(col_hbm.at[sid].at[g], colb.at[s],
                                      isems[s]),
                pltpu.make_async_copy(row_hbm.at[sid].at[g], rowb.at[s],
                                      isems[s]),
                pltpu.make_async_copy(w_hbm.at[sid].at[g], wb.at[s],
                                      isems[s]),
            ]

        def stage_idx(g, s):
            for c in idx_copies(g, s):
                c.start()

        def wait_idx(g, s):
            for c in idx_copies(g, s):
                c.wait()
            for j in range(CHUNK // 16):
                sl = pl.ds(j * 16, 16)
                colb[s, sl] = colb[s, sl] + off

        def gather_copy(g, s, b):
            return pltpu.make_async_copy(
                table_hbm.at[colb.at[s]], rows[b], gsems[b])

        def chunk_body(g, b, s, stage_ok=True, next_ok=True):
            b1 = 1 - b
            s1 = (s + 1) % 3
            s2 = (s + 2) % 3

            if stage_ok:
                stage_idx(g + 2, s2)

            if next_ok:
                wait_idx(g + 1, s1)
                pltpu.async_copy(table_hbm.at[colb.at[s1]], rows[b1],
                                 gsems[b1])

            gather_copy(g, s, b).wait()

            @pl.loop(0, CHUNK)
            def _(i):
                wv = plsc.load_gather(wb, [jnp.full((16,), s, jnp.int32),
                                           jnp.full((16,), i, jnp.int32)])
                for j in range(DT // 16):
                    sl = pl.ds(j * 16, 16)
                    rows[b][i, sl] = rows[b][i, sl] * wv

            pltpu.sync_copy(rows[b], acc_sh.at[rowb.at[s]], add=True)

        stage_idx(0, 0)
        stage_idx(1, 1)
        wait_idx(0, 0)
        pltpu.async_copy(table_hbm.at[colb.at[0]], rows[0], gsems[0])

        @pl.loop(0, NCH - 6, step=6)
        def _(g0):
            for k in range(6):
                chunk_body(g0 + k, k % 2, k % 3)

        for g in range(NCH - 6, NCH):
            chunk_body(g, g % 2, g % 3,
                       stage_ok=(g + 2 < NCH), next_ok=(g + 1 < NCH))

        plsc.subcore_barrier()

        @pl.loop(0, 640, step=CHUNK)
        def _(r):
            pltpu.sync_copy(acc_sh.at[pl.ds(stripe + r, CHUNK)],
                            out_hbm.at[cid].at[pl.ds(stripe + r, CHUNK)])

    return spmm(table, colp, rowp, wp)


def _statsB1_body(a0_ref, a1_ref, agg_ref, stats_ref, acc_ref):
    i = pl.program_id(0)
    deg = a0_ref[:, DH:DH + 1]
    deg = jnp.where(deg < 0.5, deg + 1.0, deg)
    inv = 1.0 / deg
    agg = jnp.concatenate([a0_ref[:, :DH] * inv, a1_ref[:, :DH] * inv], axis=1)
    agg_ref[...] = agg

    @pl.when(i == 0)
    def _():
        acc_ref[...] = jnp.zeros_like(acc_ref)

    acc_ref[0:1, :] += jnp.sum(agg, axis=0, keepdims=True)
    acc_ref[1:2, :] += jnp.sum(agg * agg, axis=0, keepdims=True)

    @pl.when(i == pl.num_programs(0) - 1)
    def _():
        stats_ref[...] = acc_ref[...]


def _stats(a0, a1):
    B = 1024
    grid = (N_PAD // B,)
    return pl.pallas_call(
        _statsB1_body,
        grid=grid,
        in_specs=[
            pl.BlockSpec((B, DT), lambda i: (i, 0)),
            pl.BlockSpec((B, DT), lambda i: (i, 0)),
        ],
        out_specs=[
            pl.BlockSpec((B, D), lambda i: (i, 0)),
            pl.BlockSpec((2, D), lambda i: (0, 0)),
        ],
        out_shape=[
            jax.ShapeDtypeStruct((N_PAD, D), jnp.float32),
            jax.ShapeDtypeStruct((2, D), jnp.float32),
        ],
        scratch_shapes=[pltpu.VMEM((2, D), jnp.float32)],
    )(a0, a1)


def _finalB2_body(agg_ref, x_ref, m_ref, stats_ref, wc_ref, bc_ref,
                  gs_ref, gb_ref, out_ref):
    mean = stats_ref[0:1, :] * (1.0 / N_NODES)
    ex2 = stats_ref[1:2, :] * (1.0 / N_NODES)
    var = ex2 - mean * mean
    rstd = lax.rsqrt(var + GN_EPS)
    xn = (agg_ref[...] - mean) * (rstd * gs_ref[...]) + gb_ref[...]
    z = jnp.concatenate([xn, x_ref[...]], axis=1)
    y = jnp.dot(z, wc_ref[...], precision=_HIGH) + bc_ref[...]
    y1 = y[:, :D]
    y0 = y[:, D:]
    m = m_ref[...]
    c1 = (1.0 - Z_RATIO) + (2.0 * Z_RATIO - 1.0) * m
    out_ref[...] = c1 * y1 + (1.0 - c1) * y0


def _final(agg, x_, maskf, stats, Wc, bc, gs, gb):
    B = 1000
    grid = (N_NODES // B,)
    return pl.pallas_call(
        _finalB2_body,
        grid=grid,
        in_specs=[
            pl.BlockSpec((B, D), lambda i: (i, 0)),
            pl.BlockSpec((B, D), lambda i: (i, 0)),
            pl.BlockSpec((B, 1), lambda i: (i, 0)),
            pl.BlockSpec((2, D), lambda i: (0, 0)),
            pl.BlockSpec((2 * D, 2 * D), lambda i: (0, 0)),
            pl.BlockSpec((1, 2 * D), lambda i: (0, 0)),
            pl.BlockSpec((1, D), lambda i: (0, 0)),
            pl.BlockSpec((1, D), lambda i: (0, 0)),
        ],
        out_specs=pl.BlockSpec((B, D), lambda i: (i, 0)),
        out_shape=jax.ShapeDtypeStruct((N_NODES, D), jnp.float32),
    )(agg, x_, maskf, stats, Wc, bc, gs, gb)


def kernel(x_, edge_index, edge_weight, mask, Wt0, bt0, Wt1, bt1,
           Wc0, bc0, Wc1, bc1, gn_scale, gn_bias):
    row = edge_index[0]
    col = edge_index[1]
    padn = E_PAD - E
    colp = jnp.pad(col, (0, padn)).reshape(NSUB, NCH, CHUNK)
    rowp = jnp.pad(row, (0, padn)).reshape(NSUB, NCH, CHUNK)
    wp = jnp.pad(edge_weight, (0, padn)).reshape(NSUB, NCH, CHUNK)
    maskf = mask.astype(jnp.float32)

    Wt = jnp.concatenate([Wt1, Wt0], axis=1)
    bt = jnp.concatenate([bt1, bt0]).reshape(1, 2 * D)
    Wc = jnp.concatenate([Wc1, Wc0], axis=1)
    bc = jnp.concatenate([bc1, bc0]).reshape(1, 2 * D)

    table = _build_table(x_, maskf, Wt, bt)
    acc = _sc_spmm(table.reshape(2 * N_NODES, DT), colp, rowp, wp)
    agg, stats = _stats(acc[0], acc[1])
    return _final(agg[:N_NODES], x_, maskf, stats, Wc, bc,
                  gn_scale.reshape(1, D), gn_bias.reshape(1, D))


# speedup vs baseline: 1.7138x; 1.7138x over previous
import dataclasses
import functools

import jax
import jax.numpy as jnp
import numpy as np
from jax import lax
from jax.experimental import pallas as pl
from jax.experimental.pallas import tpu as pltpu
from jax.experimental.pallas import tpu_sc as plsc

N_NODES = 10000
N_PAD = 10240
D = 256
DH = 128
DT = 144
E = 160000
CHUNK = 128
NCH = 84
NSUB = 16
NCORE = 2
EPW = NCH * CHUNK
E_PAD = EPW * NSUB
Z_RATIO = 0.8
GN_EPS = 1e-9

_HIGH = lax.Precision.HIGHEST

_PERM128 = np.empty(128, np.int64)
for _j in range(4):
    for _t in range(16):
        _PERM128[32 * _j + _t] = 32 * _j + 2 * _t
        _PERM128[32 * _j + 16 + _t] = 32 * _j + 2 * _t + 1
_PERM256 = np.concatenate([_PERM128, 128 + _PERM128])
_PERM512 = np.concatenate([_PERM256, 256 + np.arange(256)])


def _tableA_body(x_ref, m_ref, wt_ref, bt_ref, out_ref):
    x = x_ref[...]
    h = jnp.dot(x, wt_ref[...], precision=_HIGH) + bt_ref[...]
    h = jnp.maximum(h, 0.0)
    h1 = h[:, :D]
    h0 = h[:, D:]
    m = m_ref[...]
    c1 = (1.0 - Z_RATIO) + (2.0 * Z_RATIO - 1.0) * m
    xm = c1 * h1 + (1.0 - c1) * h0
    out_ref[0] = xm[:, :DH].astype(jnp.bfloat16)
    out_ref[1] = xm[:, DH:].astype(jnp.bfloat16)


def _build_table(x_, maskf, Wt, bt):
    B = 1000
    grid = (N_NODES // B,)
    return pl.pallas_call(
        _tableA_body,
        grid=grid,
        in_specs=[
            pl.BlockSpec((B, D), lambda i: (i, 0)),
            pl.BlockSpec((B, 1), lambda i: (i, 0)),
            pl.BlockSpec((D, 2 * D), lambda i: (0, 0)),
            pl.BlockSpec((1, 2 * D), lambda i: (0, 0)),
        ],
        out_specs=pl.BlockSpec((2, B, DH), lambda i: (0, i, 0)),
        out_shape=jax.ShapeDtypeStruct((2, N_NODES, DH), jnp.bfloat16),
    )(x_, maskf, Wt, bt)


def _sc_spmm(table, idxp):
    mesh = plsc.VectorSubcoreMesh(core_axis_name="c", subcore_axis_name="s")
    cp = pltpu.CompilerParams()
    if "needs_layout_passes" in pltpu.CompilerParams.__dataclass_fields__:
        cp = dataclasses.replace(cp, needs_layout_passes=False)
    if "use_tc_tiling_on_sc" in pltpu.CompilerParams.__dataclass_fields__:
        cp = dataclasses.replace(cp, use_tc_tiling_on_sc=False)

    @functools.partial(
        pl.kernel,
        out_type=jax.ShapeDtypeStruct((NCORE, N_PAD, DT), jnp.float32),
        mesh=mesh,
        compiler_params=cp,
        scratch_types=[
            pltpu.VMEM((9, CHUNK), jnp.int32),
            pltpu.VMEM((CHUNK, DH), jnp.bfloat16),
            pltpu.VMEM((CHUNK, DH), jnp.bfloat16),
            pltpu.VMEM((CHUNK, DT), jnp.float32),
            pltpu.VMEM_SHARED((N_PAD, DT), jnp.float32),
            pltpu.SemaphoreType.DMA,
            pltpu.SemaphoreType.DMA,
            pltpu.SemaphoreType.DMA,
            pltpu.SemaphoreType.DMA,
            pltpu.SemaphoreType.DMA,
        ],
    )
    def spmm(table_hbm, idx_hbm, out_hbm,
             idxb, r0, r1, fout, acc_sh,
             gs0, gs1, is0, is1, is2):
        cid = lax.axis_index("c")
        sid = lax.axis_index("s")
        rows = [r0, r1]
        gsems = [gs0, gs1]
        isems = [is0, is1, is2]
        zero16 = jnp.zeros((16,), jnp.float32)
        degmask = (lax.iota(jnp.int32, 16) == 0).astype(jnp.float32)
        off = cid * N_NODES

        @pl.loop(0, CHUNK)
        def _(r):
            for j in range(DT // 16):
                fout[r, pl.ds(j * 16, 16)] = zero16

        stripe = sid * 640

        @pl.loop(0, 640, step=CHUNK)
        def _(r):
            pltpu.sync_copy(fout, acc_sh.at[pl.ds(stripe + r, CHUNK)])

        plsc.subcore_barrier()

        def idx_copy(g, s):
            return pltpu.make_async_copy(
                idx_hbm.at[sid].at[pl.ds(3 * g, 3)],
                idxb.at[pl.ds(3 * s, 3)], isems[s])

        def wait_idx(g, s):
            idx_copy(g, s).wait()
            for j in range(CHUNK // 16):
                sl = pl.ds(j * 16, 16)
                idxb[3 * s, sl] = idxb[3 * s, sl] + off

        def gather_copy(s, b):
            return pltpu.make_async_copy(
                table_hbm.at[idxb.at[3 * s]], rows[b], gsems[b])

        def chunk_body(g, b, s, stage_ok=True, next_ok=True):
            b1 = 1 - b
            s1 = (s + 1) % 3
            s2 = (s + 2) % 3

            if stage_ok:
                idx_copy(g + 2, s2).start()

            if next_ok:
                wait_idx(g + 1, s1)
                pltpu.async_copy(table_hbm.at[idxb.at[3 * s1]], rows[b1],
                                 gsems[b1])

            gather_copy(s, b).wait()

            @pl.loop(0, CHUNK)
            def _(i):
                iv = jnp.full((16,), i, jnp.int32)
                wl = plsc.load_gather(
                    idxb, [jnp.full((16,), 3 * s + 2, jnp.int32), iv])
                wf = plsc.bitcast(wl, jnp.float32)
                for j in range(DH // 32):
                    hv = rows[b][i, pl.ds(32 * j, 32)]
                    lo, hi = plsc.unpack(hv,
                                         format=plsc.PackFormat.INTERLEAVED)
                    fout[i, pl.ds(32 * j, 16)] = lo * wf
                    fout[i, pl.ds(32 * j + 16, 16)] = hi * wf
                fout[i, pl.ds(DH, 16)] = wf * degmask

            pltpu.sync_copy(fout, acc_sh.at[idxb.at[3 * s + 1]], add=True)

        idx_copy(0, 0).start()
        idx_copy(1, 1).start()
        wait_idx(0, 0)
        pltpu.async_copy(table_hbm.at[idxb.at[0]], rows[0], gsems[0])

        @pl.loop(0, NCH - 6, step=6)
        def _(g0):
            for k in range(6):
                chunk_body(g0 + k, k % 2, k % 3)

        for g in range(NCH - 6, NCH):
            chunk_body(g, g % 2, g % 3,
                       stage_ok=(g + 2 < NCH), next_ok=(g + 1 < NCH))

        plsc.subcore_barrier()

        @pl.loop(0, 640, step=CHUNK)
        def _(r):
            pltpu.sync_copy(acc_sh.at[pl.ds(stripe + r, CHUNK)],
                            out_hbm.at[cid].at[pl.ds(stripe + r, CHUNK)])

    return spmm(table, idxp)


def _statsB1_body(a0_ref, a1_ref, agg_ref, stats_ref, acc_ref):
    i = pl.program_id(0)
    deg = a0_ref[:, DH:DH + 1]
    deg = jnp.where(deg < 0.5, deg + 1.0, deg)
    inv = 1.0 / deg
    agg = jnp.concatenate([a0_ref[:, :DH] * inv, a1_ref[:, :DH] * inv], axis=1)
    agg_ref[...] = agg

    @pl.when(i == 0)
    def _():
        acc_ref[...] = jnp.zeros_like(acc_ref)

    acc_ref[0:1, :] += jnp.sum(agg, axis=0, keepdims=True)
    acc_ref[1:2, :] += jnp.sum(agg * agg, axis=0, keepdims=True)

    @pl.when(i == pl.num_programs(0) - 1)
    def _():
        stats_ref[...] = acc_ref[...]


def _stats(a0, a1):
    B = 1024
    grid = (N_PAD // B,)
    return pl.pallas_call(
        _statsB1_body,
        grid=grid,
        in_specs=[
            pl.BlockSpec((B, DT), lambda i: (i, 0)),
            pl.BlockSpec((B, DT), lambda i: (i, 0)),
        ],
        out_specs=[
            pl.BlockSpec((B, D), lambda i: (i, 0)),
            pl.BlockSpec((2, D), lambda i: (0, 0)),
        ],
        out_shape=[
            jax.ShapeDtypeStruct((N_PAD, D), jnp.float32),
            jax.ShapeDtypeStruct((2, D), jnp.float32),
        ],
        scratch_shapes=[pltpu.VMEM((2, D), jnp.float32)],
    )(a0, a1)


def _finalB2_body(agg_ref, x_ref, m_ref, stats_ref, wc_ref, bc_ref,
                  gs_ref, gb_ref, out_ref):
    mean = stats_ref[0:1, :] * (1.0 / N_NODES)
    ex2 = stats_ref[1:2, :] * (1.0 / N_NODES)
    var = ex2 - mean * mean
    rstd = lax.rsqrt(var + GN_EPS)
    xn = (agg_ref[...] - mean) * (rstd * gs_ref[...]) + gb_ref[...]
    z = jnp.concatenate([xn, x_ref[...]], axis=1)
    y = jnp.dot(z, wc_ref[...], precision=_HIGH) + bc_ref[...]
    y1 = y[:, :D]
    y0 = y[:, D:]
    m = m_ref[...]
    c1 = (1.0 - Z_RATIO) + (2.0 * Z_RATIO - 1.0) * m
    out_ref[...] = c1 * y1 + (1.0 - c1) * y0


def _final(agg, x_, maskf, stats, Wc, bc, gs, gb):
    B = 1000
    grid = (N_NODES // B,)
    return pl.pallas_call(
        _finalB2_body,
        grid=grid,
        in_specs=[
            pl.BlockSpec((B, D), lambda i: (i, 0)),
            pl.BlockSpec((B, D), lambda i: (i, 0)),
            pl.BlockSpec((B, 1), lambda i: (i, 0)),
            pl.BlockSpec((2, D), lambda i: (0, 0)),
            pl.BlockSpec((2 * D, 2 * D), lambda i: (0, 0)),
            pl.BlockSpec((1, 2 * D), lambda i: (0, 0)),
            pl.BlockSpec((1, D), lambda i: (0, 0)),
            pl.BlockSpec((1, D), lambda i: (0, 0)),
        ],
        out_specs=pl.BlockSpec((B, D), lambda i: (i, 0)),
        out_shape=jax.ShapeDtypeStruct((N_NODES, D), jnp.float32),
    )(agg, x_, maskf, stats, Wc, bc, gs, gb)


def kernel(x_, edge_index, edge_weight, mask, Wt0, bt0, Wt1, bt1,
           Wc0, bc0, Wc1, bc1, gn_scale, gn_bias):
    row = edge_index[0]
    col = edge_index[1]
    padn = E_PAD - E
    colp = jnp.pad(col, (0, padn)).reshape(NSUB, NCH, 1, CHUNK)
    rowp = jnp.pad(row, (0, padn)).reshape(NSUB, NCH, 1, CHUNK)
    wp = jax.lax.bitcast_convert_type(
        jnp.pad(edge_weight, (0, padn)), jnp.int32
    ).reshape(NSUB, NCH, 1, CHUNK)
    idxp = jnp.concatenate([colp, rowp, wp], axis=2)
    idxp = idxp.reshape(NSUB, NCH * 3, CHUNK)
    maskf = mask.astype(jnp.float32)

    Wt = jnp.concatenate([Wt1, Wt0], axis=1)
    bt = jnp.concatenate([bt1, bt0]).reshape(1, 2 * D)
    Wc = jnp.concatenate([Wc1, Wc0], axis=1)[_PERM512, :]
    bc = jnp.concatenate([bc1, bc0]).reshape(1, 2 * D)

    table = _build_table(x_, maskf, Wt, bt)
    acc = _sc_spmm(table.reshape(2 * N_NODES, DH), idxp)
    agg, stats = _stats(acc[0], acc[1])
    return _final(agg[:N_NODES], x_, maskf, stats, Wc, bc,
                  gn_scale[_PERM256].reshape(1, D),
                  gn_bias[_PERM256].reshape(1, D))


# speedup vs baseline: 1.8292x; 1.0673x over previous
import dataclasses
import functools

import jax
import jax.numpy as jnp
import numpy as np
from jax import lax
from jax.experimental import pallas as pl
from jax.experimental.pallas import tpu as pltpu
from jax.experimental.pallas import tpu_sc as plsc

N_NODES = 10000
N_PAD = 10240
D = 256
DH = 128
DT = 144
E = 160000
CHUNK = 128
NCH = 84
NSUB = 16
NCORE = 2
EPW = NCH * CHUNK
E_PAD = EPW * NSUB
Z_RATIO = 0.8
GN_EPS = 1e-9

_HIGH = lax.Precision.DEFAULT

_PERM128 = np.empty(128, np.int64)
for _j in range(4):
    for _t in range(16):
        _PERM128[32 * _j + _t] = 32 * _j + 2 * _t
        _PERM128[32 * _j + 16 + _t] = 32 * _j + 2 * _t + 1
_PERM256 = np.concatenate([_PERM128, 128 + _PERM128])
_PERM512 = np.concatenate([_PERM256, 256 + np.arange(256)])


def _tableA_body(x_ref, m_ref, wt_ref, bt_ref, out_ref):
    x = x_ref[...]
    h = jnp.dot(x, wt_ref[...], precision=_HIGH) + bt_ref[...]
    h = jnp.maximum(h, 0.0)
    h1 = h[:, :D]
    h0 = h[:, D:]
    m = m_ref[...]
    c1 = (1.0 - Z_RATIO) + (2.0 * Z_RATIO - 1.0) * m
    xm = c1 * h1 + (1.0 - c1) * h0
    out_ref[0] = xm[:, :DH].astype(jnp.bfloat16)
    out_ref[1] = xm[:, DH:].astype(jnp.bfloat16)


def _build_table(x_, maskf, Wt, bt):
    B = 1000
    grid = (N_NODES // B,)
    return pl.pallas_call(
        _tableA_body,
        grid=grid,
        in_specs=[
            pl.BlockSpec((B, D), lambda i: (i, 0)),
            pl.BlockSpec((B, 1), lambda i: (i, 0)),
            pl.BlockSpec((D, 2 * D), lambda i: (0, 0)),
            pl.BlockSpec((1, 2 * D), lambda i: (0, 0)),
        ],
        out_specs=pl.BlockSpec((2, B, DH), lambda i: (0, i, 0)),
        out_shape=jax.ShapeDtypeStruct((2, N_NODES, DH), jnp.bfloat16),
    )(x_, maskf, Wt, bt)


def _sc_spmm(table, idxp):
    mesh = plsc.VectorSubcoreMesh(core_axis_name="c", subcore_axis_name="s")
    cp = pltpu.CompilerParams()
    if "needs_layout_passes" in pltpu.CompilerParams.__dataclass_fields__:
        cp = dataclasses.replace(cp, needs_layout_passes=False)
    if "use_tc_tiling_on_sc" in pltpu.CompilerParams.__dataclass_fields__:
        cp = dataclasses.replace(cp, use_tc_tiling_on_sc=False)

    @functools.partial(
        pl.kernel,
        out_type=jax.ShapeDtypeStruct((NCORE, N_PAD, DT), jnp.float32),
        mesh=mesh,
        compiler_params=cp,
        scratch_types=[
            pltpu.VMEM((9, CHUNK), jnp.int32),
            pltpu.VMEM((CHUNK, DH), jnp.bfloat16),
            pltpu.VMEM((CHUNK, DH), jnp.bfloat16),
            pltpu.VMEM((CHUNK, DT), jnp.float32),
            pltpu.VMEM_SHARED((N_PAD, DT), jnp.float32),
            pltpu.SemaphoreType.DMA,
            pltpu.SemaphoreType.DMA,
            pltpu.SemaphoreType.DMA,
            pltpu.SemaphoreType.DMA,
            pltpu.SemaphoreType.DMA,
        ],
    )
    def spmm(table_hbm, idx_hbm, out_hbm,
             idxb, r0, r1, fout, acc_sh,
             gs0, gs1, is0, is1, is2):
        cid = lax.axis_index("c")
        sid = lax.axis_index("s")
        rows = [r0, r1]
        gsems = [gs0, gs1]
        isems = [is0, is1, is2]
        zero16 = jnp.zeros((16,), jnp.float32)
        degmask = (lax.iota(jnp.int32, 16) == 0).astype(jnp.float32)
        off = cid * N_NODES

        @pl.loop(0, CHUNK)
        def _(r):
            for j in range(DT // 16):
                fout[r, pl.ds(j * 16, 16)] = zero16

        stripe = sid * 640

        @pl.loop(0, 640, step=CHUNK)
        def _(r):
            pltpu.sync_copy(fout, acc_sh.at[pl.ds(stripe + r, CHUNK)])

        plsc.subcore_barrier()

        def idx_copy(g, s):
            return pltpu.make_async_copy(
                idx_hbm.at[sid].at[pl.ds(3 * g, 3)],
                idxb.at[pl.ds(3 * s, 3)], isems[s])

        def wait_idx(g, s):
            idx_copy(g, s).wait()
            for j in range(CHUNK // 16):
                sl = pl.ds(j * 16, 16)
                idxb[3 * s, sl] = idxb[3 * s, sl] + off

        def gather_copy(s, b):
            return pltpu.make_async_copy(
                table_hbm.at[idxb.at[3 * s]], rows[b], gsems[b])

        def chunk_body(g, b, s, stage_ok=True, next_ok=True):
            b1 = 1 - b
            s1 = (s + 1) % 3
            s2 = (s + 2) % 3

            if stage_ok:
                idx_copy(g + 2, s2).start()

            if next_ok:
                wait_idx(g + 1, s1)
                pltpu.async_copy(table_hbm.at[idxb.at[3 * s1]], rows[b1],
                                 gsems[b1])

            gather_copy(s, b).wait()

            @pl.loop(0, CHUNK)
            def _(i):
                iv = jnp.full((16,), i, jnp.int32)
                wl = plsc.load_gather(
                    idxb, [jnp.full((16,), 3 * s + 2, jnp.int32), iv])
                wf = plsc.bitcast(wl, jnp.float32)
                for j in range(DH // 32):
                    hv = rows[b][i, pl.ds(32 * j, 32)]
                    lo, hi = plsc.unpack(hv,
                                         format=plsc.PackFormat.INTERLEAVED)
                    fout[i, pl.ds(32 * j, 16)] = lo * wf
                    fout[i, pl.ds(32 * j + 16, 16)] = hi * wf
                fout[i, pl.ds(DH, 16)] = wf * degmask

            pltpu.sync_copy(fout, acc_sh.at[idxb.at[3 * s + 1]], add=True)

        idx_copy(0, 0).start()
        idx_copy(1, 1).start()
        wait_idx(0, 0)
        pltpu.async_copy(table_hbm.at[idxb.at[0]], rows[0], gsems[0])

        @pl.loop(0, NCH - 6, step=6)
        def _(g0):
            for k in range(6):
                chunk_body(g0 + k, k % 2, k % 3)

        for g in range(NCH - 6, NCH):
            chunk_body(g, g % 2, g % 3,
                       stage_ok=(g + 2 < NCH), next_ok=(g + 1 < NCH))

        plsc.subcore_barrier()

        @pl.loop(0, 640, step=CHUNK)
        def _(r):
            pltpu.sync_copy(acc_sh.at[pl.ds(stripe + r, CHUNK)],
                            out_hbm.at[cid].at[pl.ds(stripe + r, CHUNK)])

    return spmm(table, idxp)


def _statsB1_body(a0_ref, a1_ref, agg_ref, stats_ref, acc_ref):
    i = pl.program_id(0)
    deg = a0_ref[:, DH:DH + 1]
    deg = jnp.where(deg < 0.5, deg + 1.0, deg)
    inv = 1.0 / deg
    agg = jnp.concatenate([a0_ref[:, :DH] * inv, a1_ref[:, :DH] * inv], axis=1)
    agg_ref[...] = agg

    @pl.when(i == 0)
    def _():
        acc_ref[...] = jnp.zeros_like(acc_ref)

    acc_ref[0:1, :] += jnp.sum(agg, axis=0, keepdims=True)
    acc_ref[1:2, :] += jnp.sum(agg * agg, axis=0, keepdims=True)

    @pl.when(i == pl.num_programs(0) - 1)
    def _():
        stats_ref[...] = acc_ref[...]


def _stats(a0, a1):
    B = 1024
    grid = (N_PAD // B,)
    return pl.pallas_call(
        _statsB1_body,
        grid=grid,
        in_specs=[
            pl.BlockSpec((B, DT), lambda i: (i, 0)),
            pl.BlockSpec((B, DT), lambda i: (i, 0)),
        ],
        out_specs=[
            pl.BlockSpec((B, D), lambda i: (i, 0)),
            pl.BlockSpec((2, D), lambda i: (0, 0)),
        ],
        out_shape=[
            jax.ShapeDtypeStruct((N_PAD, D), jnp.float32),
            jax.ShapeDtypeStruct((2, D), jnp.float32),
        ],
        scratch_shapes=[pltpu.VMEM((2, D), jnp.float32)],
    )(a0, a1)


def _finalB2_body(agg_ref, x_ref, m_ref, stats_ref, wc_ref, bc_ref,
                  gs_ref, gb_ref, out_ref):
    mean = stats_ref[0:1, :] * (1.0 / N_NODES)
    ex2 = stats_ref[1:2, :] * (1.0 / N_NODES)
    var = ex2 - mean * mean
    rstd = lax.rsqrt(var + GN_EPS)
    xn = (agg_ref[...] - mean) * (rstd * gs_ref[...]) + gb_ref[...]
    z = jnp.concatenate([xn, x_ref[...]], axis=1)
    y = jnp.dot(z, wc_ref[...], precision=_HIGH) + bc_ref[...]
    y1 = y[:, :D]
    y0 = y[:, D:]
    m = m_ref[...]
    c1 = (1.0 - Z_RATIO) + (2.0 * Z_RATIO - 1.0) * m
    out_ref[...] = c1 * y1 + (1.0 - c1) * y0


def _final(agg, x_, maskf, stats, Wc, bc, gs, gb):
    B = 1000
    grid = (N_NODES // B,)
    return pl.pallas_call(
        _finalB2_body,
        grid=grid,
        in_specs=[
            pl.BlockSpec((B, D), lambda i: (i, 0)),
            pl.BlockSpec((B, D), lambda i: (i, 0)),
            pl.BlockSpec((B, 1), lambda i: (i, 0)),
            pl.BlockSpec((2, D), lambda i: (0, 0)),
            pl.BlockSpec((2 * D, 2 * D), lambda i: (0, 0)),
            pl.BlockSpec((1, 2 * D), lambda i: (0, 0)),
            pl.BlockSpec((1, D), lambda i: (0, 0)),
            pl.BlockSpec((1, D), lambda i: (0, 0)),
        ],
        out_specs=pl.BlockSpec((B, D), lambda i: (i, 0)),
        out_shape=jax.ShapeDtypeStruct((N_NODES, D), jnp.float32),
    )(agg, x_, maskf, stats, Wc, bc, gs, gb)


def kernel(x_, edge_index, edge_weight, mask, Wt0, bt0, Wt1, bt1,
           Wc0, bc0, Wc1, bc1, gn_scale, gn_bias):
    row = edge_index[0]
    col = edge_index[1]
    padn = E_PAD - E
    colp = jnp.pad(col, (0, padn)).reshape(NSUB, NCH, 1, CHUNK)
    rowp = jnp.pad(row, (0, padn)).reshape(NSUB, NCH, 1, CHUNK)
    wp = jax.lax.bitcast_convert_type(
        jnp.pad(edge_weight, (0, padn)), jnp.int32
    ).reshape(NSUB, NCH, 1, CHUNK)
    idxp = jnp.concatenate([colp, rowp, wp], axis=2)
    idxp = idxp.reshape(NSUB, NCH * 3, CHUNK)
    maskf = mask.astype(jnp.float32)

    Wt = jnp.concatenate([Wt1, Wt0], axis=1)
    bt = jnp.concatenate([bt1, bt0]).reshape(1, 2 * D)
    Wc = jnp.concatenate([Wc1, Wc0], axis=1)[_PERM512, :]
    bc = jnp.concatenate([bc1, bc0]).reshape(1, 2 * D)

    table = _build_table(x_, maskf, Wt, bt)
    acc = _sc_spmm(table.reshape(2 * N_NODES, DH), idxp)
    agg, stats = _stats(acc[0], acc[1])
    return _final(agg[:N_NODES], x_, maskf, stats, Wc, bc,
                  gn_scale[_PERM256].reshape(1, D),
                  gn_bias[_PERM256].reshape(1, D))


# speedup vs baseline: 1.8402x; 1.0060x over previous
import dataclasses
import functools

import jax
import jax.numpy as jnp
import numpy as np
from jax import lax
from jax.experimental import pallas as pl
from jax.experimental.pallas import tpu as pltpu
from jax.experimental.pallas import tpu_sc as plsc

N_NODES = 10000
N_PAD = 10240
D = 256
DH = 128
DT = 144
E = 160000
CHUNK = 128
NCH = 84
NSUB = 16
NCORE = 2
EPW = NCH * CHUNK
E_PAD = EPW * NSUB
Z_RATIO = 0.8
GN_EPS = 1e-9

_HIGH = lax.Precision.DEFAULT

_PERM128 = np.empty(128, np.int64)
for _j in range(4):
    for _t in range(16):
        _PERM128[32 * _j + _t] = 32 * _j + 2 * _t
        _PERM128[32 * _j + 16 + _t] = 32 * _j + 2 * _t + 1
_PERM256 = np.concatenate([_PERM128, 128 + _PERM128])
_PERM512 = np.concatenate([_PERM256, 256 + np.arange(256)])


def _tableA_body(x_ref, m_ref, wt_ref, bt_ref, out_ref):
    x = x_ref[...]
    h = jnp.dot(x, wt_ref[...], precision=_HIGH) + bt_ref[...]
    h = jnp.maximum(h, 0.0)
    h1 = h[:, :D]
    h0 = h[:, D:]
    m = m_ref[...]
    c1 = (1.0 - Z_RATIO) + (2.0 * Z_RATIO - 1.0) * m
    xm = c1 * h1 + (1.0 - c1) * h0
    out_ref[0] = xm[:, :DH].astype(jnp.bfloat16)
    out_ref[1] = xm[:, DH:].astype(jnp.bfloat16)


def _build_table(x_, maskf, Wt, bt):
    B = 1000
    grid = (N_NODES // B,)
    return pl.pallas_call(
        _tableA_body,
        grid=grid,
        in_specs=[
            pl.BlockSpec((B, D), lambda i: (i, 0)),
            pl.BlockSpec((B, 1), lambda i: (i, 0)),
            pl.BlockSpec((D, 2 * D), lambda i: (0, 0)),
            pl.BlockSpec((1, 2 * D), lambda i: (0, 0)),
        ],
        out_specs=pl.BlockSpec((2, B, DH), lambda i: (0, i, 0)),
        out_shape=jax.ShapeDtypeStruct((2, N_NODES, DH), jnp.bfloat16),
    )(x_, maskf, Wt, bt)


def _sc_spmm(table, idxp):
    mesh = plsc.VectorSubcoreMesh(core_axis_name="c", subcore_axis_name="s")
    cp = pltpu.CompilerParams()
    if "needs_layout_passes" in pltpu.CompilerParams.__dataclass_fields__:
        cp = dataclasses.replace(cp, needs_layout_passes=False)
    if "use_tc_tiling_on_sc" in pltpu.CompilerParams.__dataclass_fields__:
        cp = dataclasses.replace(cp, use_tc_tiling_on_sc=False)

    @functools.partial(
        pl.kernel,
        out_type=jax.ShapeDtypeStruct((NCORE, N_PAD, DT), jnp.float32),
        mesh=mesh,
        compiler_params=cp,
        scratch_types=[
            pltpu.VMEM((9, CHUNK), jnp.int32),
            pltpu.VMEM((CHUNK, DH), jnp.bfloat16),
            pltpu.VMEM((CHUNK, DH), jnp.bfloat16),
            pltpu.VMEM((CHUNK, DT), jnp.float32),
            pltpu.VMEM_SHARED((N_PAD, DT), jnp.float32),
            pltpu.SemaphoreType.DMA,
            pltpu.SemaphoreType.DMA,
            pltpu.SemaphoreType.DMA,
            pltpu.SemaphoreType.DMA,
            pltpu.SemaphoreType.DMA,
        ],
    )
    def spmm(table_hbm, idx_hbm, out_hbm,
             idxb, r0, r1, fout, acc_sh,
             gs0, gs1, is0, is1, is2):
        cid = lax.axis_index("c")
        sid = lax.axis_index("s")
        rows = [r0, r1]
        gsems = [gs0, gs1]
        isems = [is0, is1, is2]
        zero16 = jnp.zeros((16,), jnp.float32)
        degmask = (lax.iota(jnp.int32, 16) == 0).astype(jnp.float32)
        off = cid * N_NODES

        @pl.loop(0, CHUNK)
        def _(r):
            for j in range(DT // 16):
                fout[r, pl.ds(j * 16, 16)] = zero16

        stripe = sid * 640

        @pl.loop(0, 640, step=CHUNK)
        def _(r):
            pltpu.sync_copy(fout, acc_sh.at[pl.ds(stripe + r, CHUNK)])

        plsc.subcore_barrier()

        def idx_copy(g, s):
            return pltpu.make_async_copy(
                idx_hbm.at[sid].at[pl.ds(3 * g, 3)],
                idxb.at[pl.ds(3 * s, 3)], isems[s])

        def wait_idx(g, s):
            idx_copy(g, s).wait()
            for j in range(CHUNK // 16):
                sl = pl.ds(j * 16, 16)
                idxb[3 * s, sl] = idxb[3 * s, sl] + off

        def gather_copy(s, b):
            return pltpu.make_async_copy(
                table_hbm.at[idxb.at[3 * s]], rows[b], gsems[b])

        def chunk_body(g, b, s, stage_ok=True, next_ok=True):
            b1 = 1 - b
            s1 = (s + 1) % 3
            s2 = (s + 2) % 3

            if stage_ok:
                idx_copy(g + 2, s2).start()

            if next_ok:
                wait_idx(g + 1, s1)
                pltpu.async_copy(table_hbm.at[idxb.at[3 * s1]], rows[b1],
                                 gsems[b1])

            gather_copy(s, b).wait()

            @pl.loop(0, CHUNK)
            def _(i):
                iv = jnp.full((16,), i, jnp.int32)
                wl = plsc.load_gather(
                    idxb, [jnp.full((16,), 3 * s + 2, jnp.int32), iv])
                wf = plsc.bitcast(wl, jnp.float32)
                for j in range(DH // 32):
                    hv = rows[b][i, pl.ds(32 * j, 32)]
                    lo, hi = plsc.unpack(hv,
                                         format=plsc.PackFormat.INTERLEAVED)
                    fout[i, pl.ds(32 * j, 16)] = lo * wf
                    fout[i, pl.ds(32 * j + 16, 16)] = hi * wf
                fout[i, pl.ds(DH, 16)] = wf * degmask

            pltpu.sync_copy(fout, acc_sh.at[idxb.at[3 * s + 1]], add=True)

        idx_copy(0, 0).start()
        idx_copy(1, 1).start()
        wait_idx(0, 0)
        pltpu.async_copy(table_hbm.at[idxb.at[0]], rows[0], gsems[0])

        @pl.loop(0, NCH - 6, step=6)
        def _(g0):
            for k in range(6):
                chunk_body(g0 + k, k % 2, k % 3)

        for g in range(NCH - 6, NCH):
            chunk_body(g, g % 2, g % 3,
                       stage_ok=(g + 2 < NCH), next_ok=(g + 1 < NCH))

        plsc.subcore_barrier()

        @pl.loop(0, 640, step=CHUNK)
        def _(r):
            pltpu.sync_copy(acc_sh.at[pl.ds(stripe + r, CHUNK)],
                            out_hbm.at[cid].at[pl.ds(stripe + r, CHUNK)])

    return spmm(table, idxp)


def _finalB_body(a0_ref, a1_ref, x_ref, m_ref, wc_ref, bc_ref,
                 gs_ref, gb_ref, out_ref, st_ref):
    ph = pl.program_id(0)
    j = pl.program_id(1)
    deg = a0_ref[:, DH:DH + 1]
    deg = jnp.where(deg < 0.5, deg + 1.0, deg)
    inv = 1.0 / deg
    agg = jnp.concatenate([a0_ref[:, :DH] * inv, a1_ref[:, :DH] * inv], axis=1)

    @pl.when((ph == 0) & (j == 0))
    def _():
        st_ref[...] = jnp.zeros_like(st_ref)

    @pl.when(ph == 0)
    def _():
        st_ref[0:1, :] += jnp.sum(agg, axis=0, keepdims=True)
        st_ref[1:2, :] += jnp.sum(agg * agg, axis=0, keepdims=True)

    @pl.when(ph == 1)
    def _():
        mean = st_ref[0:1, :] * (1.0 / N_NODES)
        ex2 = st_ref[1:2, :] * (1.0 / N_NODES)
        var = ex2 - mean * mean
        rstd = lax.rsqrt(var + GN_EPS)
        xn = (agg - mean) * (rstd * gs_ref[...]) + gb_ref[...]
        z = jnp.concatenate([xn, x_ref[...]], axis=1)
        y = jnp.dot(z, wc_ref[...], precision=_HIGH) + bc_ref[...]
        y1 = y[:, :D]
        y0 = y[:, D:]
        m = m_ref[...]
        c1 = (1.0 - Z_RATIO) + (2.0 * Z_RATIO - 1.0) * m
        out_ref[...] = c1 * y1 + (1.0 - c1) * y0


def _final(a0, a1, x_, maskf, Wc, bc, gs, gb):
    B = 1000
    grid = (2, N_NODES // B)
    return pl.pallas_call(
        _finalB_body,
        grid=grid,
        in_specs=[
            pl.BlockSpec((B, DT), lambda p, j: (j, 0)),
            pl.BlockSpec((B, DT), lambda p, j: (j, 0)),
            pl.BlockSpec((B, D), lambda p, j: (j, 0)),
            pl.BlockSpec((B, 1), lambda p, j: (j, 0)),
            pl.BlockSpec((2 * D, 2 * D), lambda p, j: (0, 0)),
            pl.BlockSpec((1, 2 * D), lambda p, j: (0, 0)),
            pl.BlockSpec((1, D), lambda p, j: (0, 0)),
            pl.BlockSpec((1, D), lambda p, j: (0, 0)),
        ],
        out_specs=pl.BlockSpec((B, D), lambda p, j: (j, 0)),
        out_shape=jax.ShapeDtypeStruct((N_NODES, D), jnp.float32),
        scratch_shapes=[pltpu.VMEM((2, D), jnp.float32)],
    )(a0, a1, x_, maskf, Wc, bc, gs, gb)


def kernel(x_, edge_index, edge_weight, mask, Wt0, bt0, Wt1, bt1,
           Wc0, bc0, Wc1, bc1, gn_scale, gn_bias):
    row = edge_index[0]
    col = edge_index[1]
    padn = E_PAD - E
    colp = jnp.pad(col, (0, padn)).reshape(NSUB, NCH, 1, CHUNK)
    rowp = jnp.pad(row, (0, padn)).reshape(NSUB, NCH, 1, CHUNK)
    wp = jax.lax.bitcast_convert_type(
        jnp.pad(edge_weight, (0, padn)), jnp.int32
    ).reshape(NSUB, NCH, 1, CHUNK)
    idxp = jnp.concatenate([colp, rowp, wp], axis=2)
    idxp = idxp.reshape(NSUB, NCH * 3, CHUNK)
    maskf = mask.astype(jnp.float32)

    Wt = jnp.concatenate([Wt1, Wt0], axis=1)
    bt = jnp.concatenate([bt1, bt0]).reshape(1, 2 * D)
    Wc = jnp.concatenate([Wc1, Wc0], axis=1)[_PERM512, :]
    bc = jnp.concatenate([bc1, bc0]).reshape(1, 2 * D)

    table = _build_table(x_, maskf, Wt, bt)
    acc = _sc_spmm(table.reshape(2 * N_NODES, DH), idxp)
    return _final(acc[0], acc[1], x_, maskf, Wc, bc,
                  gn_scale[_PERM256].reshape(1, D),
                  gn_bias[_PERM256].reshape(1, D))


# speedup vs baseline: 1.8550x; 1.0080x over previous
import dataclasses
import functools

import jax
import jax.numpy as jnp
import numpy as np
from jax import lax
from jax.experimental import pallas as pl
from jax.experimental.pallas import tpu as pltpu
from jax.experimental.pallas import tpu_sc as plsc

N_NODES = 10000
N_PAD = 10112
D = 256
DH = 128
DT = 144
E = 160000
CHUNK = 128
NCH = 84
NSUB = 16
NCORE = 2
EPW = NCH * CHUNK
E_PAD = EPW * NSUB
Z_RATIO = 0.8
GN_EPS = 1e-9

_HIGH = lax.Precision.DEFAULT

_PERM128 = np.empty(128, np.int64)
for _j in range(4):
    for _t in range(16):
        _PERM128[32 * _j + _t] = 32 * _j + 2 * _t
        _PERM128[32 * _j + 16 + _t] = 32 * _j + 2 * _t + 1
_PERM256 = np.concatenate([_PERM128, 128 + _PERM128])
_PERM512 = np.concatenate([_PERM256, 256 + np.arange(256)])


def _tableA_body(x_ref, m_ref, wt_ref, bt_ref, out_ref):
    x = x_ref[...]
    h = jnp.dot(x, wt_ref[...], precision=_HIGH) + bt_ref[...]
    h = jnp.maximum(h, 0.0)
    h1 = h[:, :D]
    h0 = h[:, D:]
    m = m_ref[...]
    c1 = (1.0 - Z_RATIO) + (2.0 * Z_RATIO - 1.0) * m
    xm = c1 * h1 + (1.0 - c1) * h0
    out_ref[0] = xm[:, :DH].astype(jnp.bfloat16)
    out_ref[1] = xm[:, DH:].astype(jnp.bfloat16)


def _build_table(x_, maskf, Wt, bt):
    B = 1000
    grid = (N_NODES // B,)
    return pl.pallas_call(
        _tableA_body,
        grid=grid,
        in_specs=[
            pl.BlockSpec((B, D), lambda i: (i, 0)),
            pl.BlockSpec((B, 1), lambda i: (i, 0)),
            pl.BlockSpec((D, 2 * D), lambda i: (0, 0)),
            pl.BlockSpec((1, 2 * D), lambda i: (0, 0)),
        ],
        out_specs=pl.BlockSpec((2, B, DH), lambda i: (0, i, 0)),
        out_shape=jax.ShapeDtypeStruct((2, N_NODES, DH), jnp.bfloat16),
    )(x_, maskf, Wt, bt)


def _sc_spmm(table, idxp):
    mesh = plsc.VectorSubcoreMesh(core_axis_name="c", subcore_axis_name="s")
    cp = pltpu.CompilerParams()
    if "needs_layout_passes" in pltpu.CompilerParams.__dataclass_fields__:
        cp = dataclasses.replace(cp, needs_layout_passes=False)
    if "use_tc_tiling_on_sc" in pltpu.CompilerParams.__dataclass_fields__:
        cp = dataclasses.replace(cp, use_tc_tiling_on_sc=False)

    @functools.partial(
        pl.kernel,
        out_type=jax.ShapeDtypeStruct((NCORE, N_PAD, DT), jnp.float32),
        mesh=mesh,
        compiler_params=cp,
        scratch_types=[
            pltpu.VMEM((36, CHUNK), jnp.int32),
            pltpu.VMEM((CHUNK, DH), jnp.bfloat16),
            pltpu.VMEM((CHUNK, DH), jnp.bfloat16),
            pltpu.VMEM((CHUNK, DT), jnp.float32),
            pltpu.VMEM_SHARED((N_PAD, DT), jnp.float32),
            pltpu.SemaphoreType.DMA,
            pltpu.SemaphoreType.DMA,
            pltpu.SemaphoreType.DMA,
            pltpu.SemaphoreType.DMA,
        ],
    )
    def spmm(table_hbm, idx_hbm, out_hbm,
             idxb, r0, r1, fout, acc_sh,
             gs0, gs1, is0, is1):
        cid = lax.axis_index("c")
        sid = lax.axis_index("s")
        rows = [r0, r1]
        gsems = [gs0, gs1]
        isems = [is0, is1]
        zero16 = jnp.zeros((16,), jnp.float32)
        degmask = (lax.iota(jnp.int32, 16) == 0).astype(jnp.float32)
        off = cid * N_NODES

        @pl.loop(0, CHUNK)
        def _(r):
            for j in range(DT // 16):
                fout[r, pl.ds(j * 16, 16)] = zero16

        stripe = sid * 632
        for t in range(4):
            pltpu.sync_copy(fout, acc_sh.at[pl.ds(stripe + 128 * t, CHUNK)])
        pltpu.sync_copy(fout.at[pl.ds(0, 120)],
                        acc_sh.at[pl.ds(stripe + 512, 120)])

        plsc.subcore_barrier()

        def grp_copy(G, slot):
            return pltpu.make_async_copy(
                idx_hbm.at[sid].at[pl.ds(18 * G, 18)],
                idxb.at[pl.ds(18 * slot, 18)], isems[slot])

        def wait_grp(G, slot):
            grp_copy(G, slot).wait()
            for k in range(6):
                for j in range(CHUNK // 16):
                    sl = pl.ds(j * 16, 16)
                    idxb[18 * slot + 3 * k, sl] = \
                        idxb[18 * slot + 3 * k, sl] + off

        def chunk_body(b, r3, next_r3):
            b1 = 1 - b
            if next_r3 is not None:
                pltpu.async_copy(table_hbm.at[idxb.at[next_r3]], rows[b1],
                                 gsems[b1])
            pltpu.make_async_copy(table_hbm.at[idxb.at[r3]], rows[b],
                                  gsems[b]).wait()

            @pl.loop(0, CHUNK)
            def _(i):
                iv = jnp.full((16,), i, jnp.int32)
                wl = plsc.load_gather(
                    idxb, [jnp.full((16,), r3 + 2, jnp.int32), iv])
                wf = plsc.bitcast(wl, jnp.float32)
                for j in range(DH // 32):
                    hv = rows[b][i, pl.ds(32 * j, 32)]
                    lo, hi = plsc.unpack(hv,
                                         format=plsc.PackFormat.INTERLEAVED)
                    fout[i, pl.ds(32 * j, 16)] = lo * wf
                    fout[i, pl.ds(32 * j + 16, 16)] = hi * wf
                fout[i, pl.ds(DH, 16)] = wf * degmask

            pltpu.sync_copy(fout, acc_sh.at[idxb.at[r3 + 1]], add=True)

        NGRP = NCH // 6

        def group_body(G, slot, do_stage, has_next):
            oslot = 1 - slot
            if do_stage:
                grp_copy(G + 1, oslot).start()
            for k in range(5):
                chunk_body(k % 2, 18 * slot + 3 * k, 18 * slot + 3 * (k + 1))
            if has_next:
                wait_grp(G + 1, oslot)
                chunk_body(1, 18 * slot + 15, 18 * oslot)
            else:
                chunk_body(1, 18 * slot + 15, None)

        grp_copy(0, 0).start()
        wait_grp(0, 0)
        pltpu.async_copy(table_hbm.at[idxb.at[0]], rows[0], gsems[0])

        @pl.loop(0, NGRP - 2, step=2)
        def _(G0):
            group_body(G0, 0, True, True)
            group_body(G0 + 1, 1, True, True)

        group_body(NGRP - 2, 0, True, True)
        group_body(NGRP - 1, 1, False, False)

        plsc.subcore_barrier()

        for t in range(4):
            pltpu.sync_copy(acc_sh.at[pl.ds(stripe + 128 * t, CHUNK)],
                            out_hbm.at[cid].at[pl.ds(stripe + 128 * t, CHUNK)])
        pltpu.sync_copy(acc_sh.at[pl.ds(stripe + 512, 120)],
                        out_hbm.at[cid].at[pl.ds(stripe + 512, 120)])

    return spmm(table, idxp)


def _finalB_body(a0_ref, a1_ref, x_ref, m_ref, wc_ref, bc_ref,
                 gs_ref, gb_ref, out_ref, st_ref):
    ph = pl.program_id(0)
    j = pl.program_id(1)
    deg = a0_ref[:, DH:DH + 1]
    deg = jnp.where(deg < 0.5, deg + 1.0, deg)
    inv = 1.0 / deg
    agg = jnp.concatenate([a0_ref[:, :DH] * inv, a1_ref[:, :DH] * inv], axis=1)

    @pl.when((ph == 0) & (j == 0))
    def _():
        st_ref[...] = jnp.zeros_like(st_ref)

    @pl.when(ph == 0)
    def _():
        st_ref[0:1, :] += jnp.sum(agg, axis=0, keepdims=True)
        st_ref[1:2, :] += jnp.sum(agg * agg, axis=0, keepdims=True)

    @pl.when(ph == 1)
    def _():
        mean = st_ref[0:1, :] * (1.0 / N_NODES)
        ex2 = st_ref[1:2, :] * (1.0 / N_NODES)
        var = ex2 - mean * mean
        rstd = lax.rsqrt(var + GN_EPS)
        xn = (agg - mean) * (rstd * gs_ref[...]) + gb_ref[...]
        z = jnp.concatenate([xn, x_ref[...]], axis=1)
        y = jnp.dot(z, wc_ref[...], precision=_HIGH) + bc_ref[...]
        y1 = y[:, :D]
        y0 = y[:, D:]
        m = m_ref[...]
        c1 = (1.0 - Z_RATIO) + (2.0 * Z_RATIO - 1.0) * m
        out_ref[...] = c1 * y1 + (1.0 - c1) * y0


def _final(a0, a1, x_, maskf, Wc, bc, gs, gb):
    B = 1000
    grid = (2, N_NODES // B)
    return pl.pallas_call(
        _finalB_body,
        grid=grid,
        in_specs=[
            pl.BlockSpec((B, DT), lambda p, j: (j, 0)),
            pl.BlockSpec((B, DT), lambda p, j: (j, 0)),
            pl.BlockSpec((B, D), lambda p, j: (j, 0)),
            pl.BlockSpec((B, 1), lambda p, j: (j, 0)),
            pl.BlockSpec((2 * D, 2 * D), lambda p, j: (0, 0)),
            pl.BlockSpec((1, 2 * D), lambda p, j: (0, 0)),
            pl.BlockSpec((1, D), lambda p, j: (0, 0)),
            pl.BlockSpec((1, D), lambda p, j: (0, 0)),
        ],
        out_specs=pl.BlockSpec((B, D), lambda p, j: (j, 0)),
        out_shape=jax.ShapeDtypeStruct((N_NODES, D), jnp.float32),
        scratch_shapes=[pltpu.VMEM((2, D), jnp.float32)],
    )(a0, a1, x_, maskf, Wc, bc, gs, gb)


def kernel(x_, edge_index, edge_weight, mask, Wt0, bt0, Wt1, bt1,
           Wc0, bc0, Wc1, bc1, gn_scale, gn_bias):
    row = edge_index[0]
    col = edge_index[1]
    padn = E_PAD - E
    colp = jnp.pad(col, (0, padn)).reshape(NSUB, NCH, 1, CHUNK)
    rowp = jnp.pad(row, (0, padn)).reshape(NSUB, NCH, 1, CHUNK)
    wp = jax.lax.bitcast_convert_type(
        jnp.pad(edge_weight, (0, padn)), jnp.int32
    ).reshape(NSUB, NCH, 1, CHUNK)
    idxp = jnp.concatenate([colp, rowp, wp], axis=2)
    idxp = idxp.reshape(NSUB, NCH * 3, CHUNK)
    maskf = mask.astype(jnp.float32)

    Wt = jnp.concatenate([Wt1, Wt0], axis=1)
    bt = jnp.concatenate([bt1, bt0]).reshape(1, 2 * D)
    Wc = jnp.concatenate([Wc1, Wc0], axis=1)[_PERM512, :]
    bc = jnp.concatenate([bc1, bc0]).reshape(1, 2 * D)

    table = _build_table(x_, maskf, Wt, bt)
    acc = _sc_spmm(table.reshape(2 * N_NODES, DH), idxp)
    return _final(acc[0], acc[1], x_, maskf, Wc, bc,
                  gn_scale[_PERM256].reshape(1, D),
                  gn_bias[_PERM256].reshape(1, D))


# speedup vs baseline: 1.8762x; 1.0114x over previous
import dataclasses
import functools

import jax
import jax.numpy as jnp
import numpy as np
from jax import lax
from jax.experimental import pallas as pl
from jax.experimental.pallas import tpu as pltpu
from jax.experimental.pallas import tpu_sc as plsc

N_NODES = 10000
N_PAD = 10112
D = 256
DH = 128
DT = 144
E = 160000
CHUNK = 128
NCH = 84
NSUB = 16
NCORE = 2
EPW = NCH * CHUNK
E_PAD = EPW * NSUB
Z_RATIO = 0.8
GN_EPS = 1e-9

_HIGH = lax.Precision.DEFAULT

_PERM128 = np.empty(128, np.int64)
for _j in range(4):
    for _t in range(16):
        _PERM128[32 * _j + _t] = 32 * _j + 2 * _t
        _PERM128[32 * _j + 16 + _t] = 32 * _j + 2 * _t + 1
_PERM256 = np.concatenate([_PERM128, 128 + _PERM128])
_PERM512 = np.concatenate([_PERM256, 256 + np.arange(256)])


def _tableA_body(x_ref, m_ref, wt_ref, bt_ref, out_ref):
    x = x_ref[...]
    h = jnp.dot(x, wt_ref[...], precision=_HIGH) + bt_ref[...]
    h = jnp.maximum(h, 0.0)
    h1 = h[:, :D]
    h0 = h[:, D:]
    m = m_ref[...]
    c1 = (1.0 - Z_RATIO) + (2.0 * Z_RATIO - 1.0) * m
    xm = c1 * h1 + (1.0 - c1) * h0
    out_ref[0] = xm[:, :DH].astype(jnp.bfloat16)
    out_ref[1] = xm[:, DH:].astype(jnp.bfloat16)


def _build_table(x_, maskf, Wt, bt):
    B = 1000
    grid = (N_NODES // B,)
    return pl.pallas_call(
        _tableA_body,
        grid=grid,
        in_specs=[
            pl.BlockSpec((B, D), lambda i: (i, 0)),
            pl.BlockSpec((B, 1), lambda i: (i, 0)),
            pl.BlockSpec((D, 2 * D), lambda i: (0, 0)),
            pl.BlockSpec((1, 2 * D), lambda i: (0, 0)),
        ],
        out_specs=pl.BlockSpec((2, B, DH), lambda i: (0, i, 0)),
        out_shape=jax.ShapeDtypeStruct((2, N_NODES, DH), jnp.bfloat16),
    )(x_, maskf, Wt, bt)


def _sc_spmm(table, idxp):
    mesh = plsc.VectorSubcoreMesh(core_axis_name="c", subcore_axis_name="s")
    cp = pltpu.CompilerParams()
    if "needs_layout_passes" in pltpu.CompilerParams.__dataclass_fields__:
        cp = dataclasses.replace(cp, needs_layout_passes=False)
    if "use_tc_tiling_on_sc" in pltpu.CompilerParams.__dataclass_fields__:
        cp = dataclasses.replace(cp, use_tc_tiling_on_sc=False)

    @functools.partial(
        pl.kernel,
        out_type=jax.ShapeDtypeStruct((NCORE, N_PAD, DT), jnp.float32),
        mesh=mesh,
        compiler_params=cp,
        scratch_types=[
            pltpu.VMEM((36, CHUNK), jnp.int32),
            pltpu.VMEM((CHUNK, DH), jnp.bfloat16),
            pltpu.VMEM((CHUNK, DH), jnp.bfloat16),
            pltpu.VMEM((CHUNK, DT), jnp.float32),
            pltpu.VMEM_SHARED((N_PAD, DT), jnp.float32),
            pltpu.SemaphoreType.DMA,
            pltpu.SemaphoreType.DMA,
            pltpu.SemaphoreType.DMA,
            pltpu.SemaphoreType.DMA,
        ],
    )
    def spmm(table_hbm, idx_hbm, out_hbm,
             idxb, r0, r1, fout, acc_sh,
             gs0, gs1, is0, is1):
        cid = lax.axis_index("c")
        sid = lax.axis_index("s")
        rows = [r0, r1]
        gsems = [gs0, gs1]
        isems = [is0, is1]
        zero16 = jnp.zeros((16,), jnp.float32)
        degmask = (lax.iota(jnp.int32, 16) == 0).astype(jnp.float32)
        off = cid * N_NODES

        @pl.loop(0, CHUNK)
        def _(r):
            for j in range(DT // 16):
                fout[r, pl.ds(j * 16, 16)] = zero16

        stripe = sid * 632
        for t in range(4):
            pltpu.sync_copy(fout, acc_sh.at[pl.ds(stripe + 128 * t, CHUNK)])
        pltpu.sync_copy(fout.at[pl.ds(0, 120)],
                        acc_sh.at[pl.ds(stripe + 512, 120)])

        plsc.subcore_barrier()

        def grp_copy(G, slot):
            return pltpu.make_async_copy(
                idx_hbm.at[sid].at[pl.ds(18 * G, 18)],
                idxb.at[pl.ds(18 * slot, 18)], isems[slot])

        def wait_grp(G, slot):
            grp_copy(G, slot).wait()
            for k in range(6):
                for j in range(CHUNK // 16):
                    sl = pl.ds(j * 16, 16)
                    idxb[18 * slot + 3 * k, sl] = \
                        idxb[18 * slot + 3 * k, sl] + off

        def chunk_body(b, r3, next_r3):
            b1 = 1 - b
            if next_r3 is not None:
                pltpu.async_copy(table_hbm.at[idxb.at[next_r3]], rows[b1],
                                 gsems[b1])
            pltpu.make_async_copy(table_hbm.at[idxb.at[r3]], rows[b],
                                  gsems[b]).wait()

            @pl.loop(0, CHUNK)
            def _(i):
                iv = jnp.full((16,), i, jnp.int32)
                wl = plsc.load_gather(
                    idxb, [jnp.full((16,), r3 + 2, jnp.int32), iv])
                wf = plsc.bitcast(wl, jnp.float32)
                for j in range(DH // 32):
                    hv = rows[b][i, pl.ds(32 * j, 32)]
                    lo, hi = plsc.unpack(hv,
                                         format=plsc.PackFormat.INTERLEAVED)
                    fout[i, pl.ds(32 * j, 16)] = lo * wf
                    fout[i, pl.ds(32 * j + 16, 16)] = hi * wf
                fout[i, pl.ds(DH, 16)] = wf * degmask

            pltpu.sync_copy(fout, acc_sh.at[idxb.at[r3 + 1]], add=True)

        NGRP = NCH // 6

        def group_body(G, slot, do_stage, has_next):
            oslot = 1 - slot
            if do_stage:
                grp_copy(G + 1, oslot).start()
            for k in range(5):
                chunk_body(k % 2, 18 * slot + 3 * k, 18 * slot + 3 * (k + 1))
            if has_next:
                wait_grp(G + 1, oslot)
                chunk_body(1, 18 * slot + 15, 18 * oslot)
            else:
                chunk_body(1, 18 * slot + 15, None)

        grp_copy(0, 0).start()
        wait_grp(0, 0)
        pltpu.async_copy(table_hbm.at[idxb.at[0]], rows[0], gsems[0])

        @pl.loop(0, NGRP - 2, step=2)
        def _(G0):
            group_body(G0, 0, True, True)
            group_body(G0 + 1, 1, True, True)

        group_body(NGRP - 2, 0, True, True)
        group_body(NGRP - 1, 1, False, False)

        plsc.subcore_barrier()

        for t in range(4):
            pltpu.sync_copy(acc_sh.at[pl.ds(stripe + 128 * t, CHUNK)],
                            out_hbm.at[cid].at[pl.ds(stripe + 128 * t, CHUNK)])
        pltpu.sync_copy(acc_sh.at[pl.ds(stripe + 512, 120)],
                        out_hbm.at[cid].at[pl.ds(stripe + 512, 120)])

    return spmm(table, idxp)


def _finalB_body(a0_ref, a1_ref, x_ref, m_ref, wc_ref, bc_ref,
                 gs_ref, gb_ref, out_ref, st_ref):
    ph = pl.program_id(0)
    j = pl.program_id(1)
    deg = a0_ref[:, DH:DH + 1]
    deg = jnp.where(deg < 0.5, deg + 1.0, deg)
    inv = 1.0 / deg
    agg = jnp.concatenate([a0_ref[:, :DH] * inv, a1_ref[:, :DH] * inv], axis=1)

    @pl.when((ph == 0) & (j == 0))
    def _():
        st_ref[...] = jnp.zeros_like(st_ref)

    @pl.when(ph == 0)
    def _():
        st_ref[0:1, :] += jnp.sum(agg, axis=0, keepdims=True)
        st_ref[1:2, :] += jnp.sum(agg * agg, axis=0, keepdims=True)

    @pl.when(ph == 1)
    def _():
        mean = st_ref[0:1, :] * (1.0 / N_NODES)
        ex2 = st_ref[1:2, :] * (1.0 / N_NODES)
        var = ex2 - mean * mean
        rstd = lax.rsqrt(var + GN_EPS)
        xn = (agg - mean) * (rstd * gs_ref[...]) + gb_ref[...]
        z = jnp.concatenate([xn, x_ref[...]], axis=1)
        y = jnp.dot(z, wc_ref[...], precision=_HIGH) + bc_ref[...]
        y1 = y[:, :D]
        y0 = y[:, D:]
        m = m_ref[...]
        c1 = (1.0 - Z_RATIO) + (2.0 * Z_RATIO - 1.0) * m
        out_ref[...] = c1 * y1 + (1.0 - c1) * y0


def _final(a0, a1, x_, maskf, Wc, bc, gs, gb):
    B = 1000
    grid = (2, N_NODES // B)
    return pl.pallas_call(
        _finalB_body,
        grid=grid,
        in_specs=[
            pl.BlockSpec((B, DT), lambda p, j: (j, 0)),
            pl.BlockSpec((B, DT), lambda p, j: (j, 0)),
            pl.BlockSpec((B, D), lambda p, j: (j, 0)),
            pl.BlockSpec((B, 1), lambda p, j: (j, 0)),
            pl.BlockSpec((2 * D, 2 * D), lambda p, j: (0, 0)),
            pl.BlockSpec((1, 2 * D), lambda p, j: (0, 0)),
            pl.BlockSpec((1, D), lambda p, j: (0, 0)),
            pl.BlockSpec((1, D), lambda p, j: (0, 0)),
        ],
        out_specs=pl.BlockSpec((B, D), lambda p, j: (j, 0)),
        out_shape=jax.ShapeDtypeStruct((N_NODES, D), jnp.float32),
        scratch_shapes=[pltpu.VMEM((2, D), jnp.float32)],
    )(a0, a1, x_, maskf, Wc, bc, gs, gb)


def kernel(x_, edge_index, edge_weight, mask, Wt0, bt0, Wt1, bt1,
           Wc0, bc0, Wc1, bc1, gn_scale, gn_bias):
    padn = E_PAD - E
    wbits = lax.bitcast_convert_type(edge_weight, jnp.int32)
    crw = jnp.pad(jnp.stack([edge_index[1], edge_index[0], wbits]),
                  ((0, 0), (0, padn)))
    idxp = crw.reshape(3, NSUB, NCH, CHUNK).transpose(1, 2, 0, 3)
    idxp = idxp.reshape(NSUB, NCH * 3, CHUNK)
    maskf = mask.astype(jnp.float32)

    Wt = jnp.concatenate([Wt1, Wt0], axis=1)
    bt = jnp.concatenate([bt1, bt0]).reshape(1, 2 * D)
    Wc = jnp.concatenate([Wc1, Wc0], axis=1)[_PERM512, :]
    bc = jnp.concatenate([bc1, bc0]).reshape(1, 2 * D)

    table = _build_table(x_, maskf, Wt, bt)
    acc = _sc_spmm(table.reshape(2 * N_NODES, DH), idxp)
    return _final(acc[0], acc[1], x_, maskf, Wc, bc,
                  gn_scale[_PERM256].reshape(1, D),
                  gn_bias[_PERM256].reshape(1, D))
